# baseline (speedup 1.0000x reference)
import functools

import jax
import jax.numpy as jnp
from jax import lax
from jax.experimental import pallas as pl
from jax.experimental.pallas import tpu as pltpu
from jax.experimental.pallas import tpu_sc as plsc

B, S, D = 1024, 200, 64
N = B * S
NC, NS = 2, 16
NW = NC * NS
ROWS_PER_W = N // NW
CHUNK = 128
NCHUNK = ROWS_PER_W // CHUNK
POSD = 2 * S
EPS = 1e-12
NLANE = 16
NVR = D // NLANE


def _vrsqrt(v):
    i = plsc.bitcast(v, jnp.int32)
    magic = jnp.full((NLANE,), 0x5F3759DF, dtype=jnp.int32)
    y = plsc.bitcast(magic - (i >> 1), jnp.float32)
    half = jnp.full((NLANE,), 0.5, dtype=jnp.float32)
    three_half = jnp.full((NLANE,), 1.5, dtype=jnp.float32)
    hv = half * v
    for _ in range(3):
        y = y * (three_half - hv * y * y)
    return y


def _body(idx_hbm, table_hbm, pos_hbm, gamma_hbm, beta_hbm, out_hbm,
          idx_v, pos_v, buf_v, gam_v, bet_v, sem):
    wid = lax.axis_index("s") * NC + lax.axis_index("c")
    base = wid * ROWS_PER_W
    pltpu.sync_copy(idx_hbm.at[pl.ds(base, ROWS_PER_W)], idx_v)
    pltpu.sync_copy(pos_hbm, pos_v)
    pltpu.sync_copy(gamma_hbm, gam_v)
    pltpu.sync_copy(beta_hbm, bet_v)
    g = [gam_v[pl.ds(NLANE * k, NLANE)] for k in range(NVR)]
    bt = [bet_v[pl.ds(NLANE * k, NLANE)] for k in range(NVR)]
    inv_d = jnp.float32(1.0 / D)

    def chunk_body(c, carry):
        pltpu.async_copy(
            table_hbm.at[idx_v.at[pl.ds(c * CHUNK, CHUNK)]], buf_v, sem
        ).wait()
        p0 = (c * CHUNK) % S

        def row_body(i, carry2):
            row = buf_v.at[i]
            prow = pos_v.at[p0 + i]
            e = [row[pl.ds(NLANE * k, NLANE)] + prow[pl.ds(NLANE * k, NLANE)]
                 for k in range(NVR)]
            s = (e[0] + e[1]) + (e[2] + e[3])
            q = (e[0] * e[0] + e[1] * e[1]) + (e[2] * e[2] + e[3] * e[3])
            mean = jnp.sum(s) * inv_d
            var = jnp.sum(q) * inv_d - mean * mean
            meanv = jnp.full((NLANE,), mean, dtype=jnp.float32)
            varv = jnp.full((NLANE,), var + EPS, dtype=jnp.float32)
            rv = _vrsqrt(varv)
            for k in range(NVR):
                row[pl.ds(NLANE * k, NLANE)] = (e[k] - meanv) * rv * g[k] + bt[k]
            return carry2

        lax.fori_loop(0, CHUNK, row_body, 0, unroll=2)
        pltpu.sync_copy(buf_v, out_hbm.at[pl.ds(base + c * CHUNK, CHUNK)])
        return carry

    lax.fori_loop(0, NCHUNK, chunk_body, 0)


@jax.jit
def _sc_embed(idx, table, posdbl, gamma, beta):
    mesh = plsc.VectorSubcoreMesh(core_axis_name="c", subcore_axis_name="s")
    fn = pl.kernel(
        _body,
        out_type=jax.ShapeDtypeStruct((N, D), jnp.float32),
        mesh=mesh,
        compiler_params=pltpu.CompilerParams(
            needs_layout_passes=False, use_tc_tiling_on_sc=False
        ),
        scratch_types=[
            pltpu.VMEM((ROWS_PER_W,), jnp.int32),
            pltpu.VMEM((POSD, D), jnp.float32),
            pltpu.VMEM((CHUNK, D), jnp.float32),
            pltpu.VMEM((D,), jnp.float32),
            pltpu.VMEM((D,), jnp.float32),
            pltpu.SemaphoreType.DMA,
        ],
    )
    return fn(idx, table, posdbl, gamma, beta)


def kernel(x, W_word, W_pos, gamma, beta):
    idx = x.reshape(N).astype(jnp.int32)
    posdbl = jnp.concatenate([W_pos, W_pos], axis=0)
    out = _sc_embed(idx, W_word, posdbl,
                    gamma.astype(jnp.float32), beta.astype(jnp.float32))
    return out.reshape(B, S, D)


# speedup vs baseline: 1.3169x; 1.3169x over previous
import functools

import jax
import jax.numpy as jnp
from jax import lax
from jax.experimental import pallas as pl
from jax.experimental.pallas import tpu as pltpu
from jax.experimental.pallas import tpu_sc as plsc

B, S, D = 1024, 200, 64
N = B * S
NC, NS = 2, 16
NW = NC * NS
ROWS_PER_W = N // NW
CHUNK = 64
NCHUNK = ROWS_PER_W // CHUNK
NBUF = 4
POSD = 2 * S
EPS = 1e-12
NLANE = 16
NVR = D // NLANE
UNROLL = 8


def _srsqrt(v):
    i = lax.bitcast_convert_type(v, jnp.int32)
    y = lax.bitcast_convert_type(jnp.int32(0x5F3759DF) - (i >> 1), jnp.float32)
    hv = jnp.float32(0.5) * v
    for _ in range(3):
        y = y * (jnp.float32(1.5) - hv * y * y)
    return y


def _body(idx_hbm, table_hbm, pos_hbm, gamma_hbm, beta_hbm, out_hbm,
          idx_v, pos_v, bufs, gam_v, bet_v, gsems, ssems):
    wid = lax.axis_index("s") * NC + lax.axis_index("c")
    base = wid * ROWS_PER_W
    pltpu.sync_copy(idx_hbm.at[pl.ds(base, ROWS_PER_W)], idx_v)
    pltpu.sync_copy(pos_hbm, pos_v)
    pltpu.sync_copy(gamma_hbm, gam_v)
    pltpu.sync_copy(beta_hbm, bet_v)
    g = [gam_v[pl.ds(NLANE * k, NLANE)] for k in range(NVR)]
    bt = [bet_v[pl.ds(NLANE * k, NLANE)] for k in range(NVR)]
    inv_d = jnp.float32(1.0 / D)

    def gather(c, b):
        return pltpu.make_async_copy(
            table_hbm.at[idx_v.at[pl.ds(c * CHUNK, CHUNK)]], bufs[b], gsems[b])

    def store(c, b):
        return pltpu.make_async_copy(
            bufs[b], out_hbm.at[pl.ds(base + c * CHUNK, CHUNK)], ssems[b])

    def compute_chunk(c, buf):
        p0 = (c * CHUNK) % S

        @plsc.parallel_loop(0, CHUNK, unroll=UNROLL)
        def row_body(i):
            row = buf.at[i]
            prow = pos_v.at[p0 + i]
            e = [row[pl.ds(NLANE * k, NLANE)] + prow[pl.ds(NLANE * k, NLANE)]
                 for k in range(NVR)]
            s = (e[0] + e[1]) + (e[2] + e[3])
            q = (e[0] * e[0] + e[1] * e[1]) + (e[2] * e[2] + e[3] * e[3])
            mean = jnp.sum(s) * inv_d
            var = jnp.sum(q) * inv_d - mean * mean + jnp.float32(EPS)
            a = _srsqrt(var)
            nb = -mean * a
            av = jnp.full((NLANE,), a, dtype=jnp.float32)
            bv = jnp.full((NLANE,), nb, dtype=jnp.float32)
            for k in range(NVR):
                row[pl.ds(NLANE * k, NLANE)] = (e[k] * av + bv) * g[k] + bt[k]

    gather(0, 0).start()

    def outer(gi, carry):
        for b0 in range(NBUF):
            c = gi * NBUF + b0
            bnext = (b0 + 1) % NBUF

            @pl.when(c >= NBUF - 1)
            def _():
                store(c - (NBUF - 1), bnext).wait()

            @pl.when(c <= NCHUNK - 2)
            def _():
                gather(c + 1, bnext).start()

            gather(c, b0).wait()
            compute_chunk(c, bufs[b0])
            store(c, b0).start()
        return carry

    lax.fori_loop(0, NCHUNK // NBUF, outer, 0)
    for j in range(NBUF - 1, 0, -1):
        c = NCHUNK - j
        store(c, c % NBUF).wait()


@jax.jit
def _sc_embed(idx, table, posdbl, gamma, beta):
    mesh = plsc.VectorSubcoreMesh(core_axis_name="c", subcore_axis_name="s")
    fn = pl.kernel(
        _body,
        out_type=jax.ShapeDtypeStruct((N, D), jnp.float32),
        mesh=mesh,
        compiler_params=pltpu.CompilerParams(
            needs_layout_passes=False, use_tc_tiling_on_sc=False
        ),
        scratch_types=[
            pltpu.VMEM((ROWS_PER_W,), jnp.int32),
            pltpu.VMEM((POSD, D), jnp.float32),
            [pltpu.VMEM((CHUNK, D), jnp.float32) for _ in range(NBUF)],
            pltpu.VMEM((D,), jnp.float32),
            pltpu.VMEM((D,), jnp.float32),
            [pltpu.SemaphoreType.DMA for _ in range(NBUF)],
            [pltpu.SemaphoreType.DMA for _ in range(NBUF)],
        ],
    )
    return fn(idx, table, posdbl, gamma, beta)


def kernel(x, W_word, W_pos, gamma, beta):
    idx = x.reshape(N).astype(jnp.int32)
    posdbl = jnp.concatenate([W_pos, W_pos], axis=0)
    out = _sc_embed(idx, W_word, posdbl,
                    gamma.astype(jnp.float32), beta.astype(jnp.float32))
    return out.reshape(B, S, D)


# speedup vs baseline: 1.3806x; 1.0484x over previous
import jax
import jax.numpy as jnp
from jax import lax
from jax.experimental import pallas as pl
from jax.experimental.pallas import tpu as pltpu
from jax.experimental.pallas import tpu_sc as plsc

B, S, D = 1024, 200, 64
VOCAB = 1000000
N = B * S
NC, NS = 2, 16
NW = NC * NS
NCHUNK = B // NW
CHUNK = S
G1 = 128
G2 = CHUNK - G1
EPS = 1e-12
NLANE = 16
NVR = D // NLANE
UNROLL = 8


def _srsqrt(v):
    i = lax.bitcast_convert_type(v, jnp.int32)
    y = lax.bitcast_convert_type(jnp.int32(0x5F3759DF) - (i >> 1), jnp.float32)
    hv = jnp.float32(0.5) * v
    for _ in range(3):
        y = y * (jnp.float32(1.5) - hv * y * y)
    return y


def _body(idxp_hbm, hoff_hbm, table2_hbm, pos_hbm, gamma_hbm, beta_hbm,
          out3_hbm, idxp_v, hoff_v, pos_v, pbufs, obufs, gam_v, bet_v,
          gsems, ssems):
    wid = lax.axis_index("s") * NC + lax.axis_index("c")
    base = wid * NCHUNK * CHUNK
    brow = wid * NCHUNK
    pltpu.sync_copy(idxp_hbm.at[pl.ds(base, NCHUNK * CHUNK)], idxp_v)
    pltpu.sync_copy(hoff_hbm.at[pl.ds(base, NCHUNK * CHUNK)],
                    hoff_v.at[pl.ds(0, NCHUNK * CHUNK)])
    pltpu.sync_copy(pos_hbm, pos_v)
    pltpu.sync_copy(gamma_hbm, gam_v)
    pltpu.sync_copy(beta_hbm, bet_v)
    g = [gam_v[pl.ds(NLANE * k, NLANE)] for k in range(NVR)]
    bt = [bet_v[pl.ds(NLANE * k, NLANE)] for k in range(NVR)]
    inv_d = jnp.float32(1.0 / D)

    def gather(c, b):
        return (
            pltpu.make_async_copy(
                table2_hbm.at[idxp_v.at[pl.ds(c * CHUNK, G1)]],
                pbufs[b].at[pl.ds(0, G1)], gsems[b]),
            pltpu.make_async_copy(
                table2_hbm.at[idxp_v.at[pl.ds(c * CHUNK + G1, G2)]],
                pbufs[b].at[pl.ds(G1, G2)], gsems[b]),
        )

    def store(c, b):
        return pltpu.make_async_copy(obufs[b], out3_hbm.at[brow + c], ssems[b])

    def compute_chunk(c, pbuf, obuf):
        coff = c * CHUNK

        @plsc.parallel_loop(0, CHUNK, unroll=UNROLL)
        def row_body(i):
            off = hoff_v[pl.ds(coff + i, NLANE)][0]
            poff = i * D
            e = [pbuf[i, pl.ds(off + NLANE * k, NLANE)] +
                 pos_v[pl.ds(poff + NLANE * k, NLANE)]
                 for k in range(NVR)]
            s = (e[0] + e[1]) + (e[2] + e[3])
            q = (e[0] * e[0] + e[1] * e[1]) + (e[2] * e[2] + e[3] * e[3])
            mean = jnp.sum(s) * inv_d
            var = jnp.sum(q) * inv_d - mean * mean + jnp.float32(EPS)
            a = _srsqrt(var)
            nb = -mean * a
            av = jnp.full((NLANE,), a, dtype=jnp.float32)
            bv = jnp.full((NLANE,), nb, dtype=jnp.float32)
            for k in range(NVR):
                obuf[i, pl.ds(NLANE * k, NLANE)] = (e[k] * av + bv) * g[k] + bt[k]

    for d in gather(0, 0):
        d.start()

    def outer(gi, carry):
        for b in range(2):
            c = gi * 2 + b

            @pl.when(c <= NCHUNK - 2)
            def _():
                for d in gather(c + 1, 1 - b):
                    d.start()

            for d in gather(c, b):
                d.wait()

            @pl.when(c >= 2)
            def _():
                store(c - 2, b).wait()

            compute_chunk(c, pbufs[b], obufs[b])
            store(c, b).start()
        return carry

    lax.fori_loop(0, NCHUNK // 2, outer, 0)
    store(NCHUNK - 2, 0).wait()
    store(NCHUNK - 1, 1).wait()


@jax.jit
def _sc_embed(idxp, hoff, table2, pos_flat, gamma, beta):
    mesh = plsc.VectorSubcoreMesh(core_axis_name="c", subcore_axis_name="s")
    fn = pl.kernel(
        _body,
        out_type=jax.ShapeDtypeStruct((B, S, D), jnp.float32),
        mesh=mesh,
        compiler_params=pltpu.CompilerParams(
            needs_layout_passes=False, use_tc_tiling_on_sc=False
        ),
        scratch_types=[
            pltpu.VMEM((NCHUNK * CHUNK,), jnp.int32),
            pltpu.VMEM((NCHUNK * CHUNK + NLANE,), jnp.int32),
            pltpu.VMEM((S * D,), jnp.float32),
            [pltpu.VMEM((CHUNK, 2 * D), jnp.float32) for _ in range(2)],
            [pltpu.VMEM((CHUNK, D), jnp.float32) for _ in range(2)],
            pltpu.VMEM((D,), jnp.float32),
            pltpu.VMEM((D,), jnp.float32),
            [pltpu.SemaphoreType.DMA for _ in range(2)],
            [pltpu.SemaphoreType.DMA for _ in range(2)],
        ],
    )
    return fn(idxp, hoff, table2, pos_flat, gamma, beta)


@jax.jit
def _prepack_table(W_word):
    return W_word.reshape(VOCAB // 2, 2 * D)


@jax.jit
def _prepack_pos(W_pos):
    return W_pos.reshape(S * D)


_prepack_cache = []


def _prepacked(W_word, W_pos):
    for ent in _prepack_cache:
        if ent[0] is W_word and ent[1] is W_pos:
            return ent[2], ent[3]
    table2 = _prepack_table(W_word)
    pos_flat = _prepack_pos(W_pos)
    _prepack_cache.insert(0, (W_word, W_pos, table2, pos_flat))
    del _prepack_cache[4:]
    return table2, pos_flat


def kernel(x, W_word, W_pos, gamma, beta):
    xi = x.reshape(N).astype(jnp.int32)
    idxp = xi >> 1
    hoff = (xi & 1) << 6
    table2, pos_flat = _prepacked(W_word, W_pos)
    return _sc_embed(idxp, hoff, table2, pos_flat,
                     gamma.astype(jnp.float32), beta.astype(jnp.float32))


# speedup vs baseline: 1.3931x; 1.0090x over previous
import jax
import jax.numpy as jnp
from jax import lax
from jax.experimental import pallas as pl
from jax.experimental.pallas import tpu as pltpu
from jax.experimental.pallas import tpu_sc as plsc

B, S, D = 1024, 200, 64
VOCAB = 1000000
N = B * S
NC, NS = 2, 16
NW = NC * NS
NCHUNK = B // NW
CHUNK = S
G1 = 128
G2 = CHUNK - G1
EPS = 1e-12
NLANE = 16
NVR = D // NLANE
UNROLL = 8


def _srsqrt(v):
    i = lax.bitcast_convert_type(v, jnp.int32)
    y = lax.bitcast_convert_type(jnp.int32(0x5F3759DF) - (i >> 1), jnp.float32)
    hv = jnp.float32(0.5) * v
    for _ in range(3):
        y = y * (jnp.float32(1.5) - hv * y * y)
    return y


def _body(idx_hbm, table_hbm, pos_hbm, gamma_hbm, beta_hbm,
          out3_hbm, idx_v, pos_v, pbufs, obufs, gam_v, bet_v,
          gsems, ssems):
    wid = lax.axis_index("s") * NC + lax.axis_index("c")
    base = wid * NCHUNK * CHUNK
    brow = wid * NCHUNK
    pltpu.sync_copy(idx_hbm.at[pl.ds(base, NCHUNK * CHUNK)], idx_v)
    pltpu.sync_copy(pos_hbm, pos_v)
    pltpu.sync_copy(gamma_hbm, gam_v)
    pltpu.sync_copy(beta_hbm, bet_v)
    g = [gam_v[pl.ds(NLANE * k, NLANE)] for k in range(NVR)]
    bt = [bet_v[pl.ds(NLANE * k, NLANE)] for k in range(NVR)]
    inv_d = jnp.float32(1.0 / D)

    def gather(c, b):
        return (
            pltpu.make_async_copy(
                table_hbm.at[idx_v.at[pl.ds(c * CHUNK, G1)]],
                pbufs[b].at[pl.ds(0, G1)], gsems[b]),
            pltpu.make_async_copy(
                table_hbm.at[idx_v.at[pl.ds(c * CHUNK + G1, G2)]],
                pbufs[b].at[pl.ds(G1, G2)], gsems[b]),
        )

    def store(c, b):
        return pltpu.make_async_copy(obufs[b], out3_hbm.at[brow + c], ssems[b])

    def compute_chunk(pbuf, obuf):
        @plsc.parallel_loop(0, CHUNK, unroll=UNROLL)
        def row_body(i):
            poff = i * D
            e = [pbuf[i, pl.ds(NLANE * k, NLANE)] +
                 pos_v[pl.ds(poff + NLANE * k, NLANE)]
                 for k in range(NVR)]
            s = (e[0] + e[1]) + (e[2] + e[3])
            q = (e[0] * e[0] + e[1] * e[1]) + (e[2] * e[2] + e[3] * e[3])
            mean = jnp.sum(s) * inv_d
            var = jnp.sum(q) * inv_d - mean * mean + jnp.float32(EPS)
            a = _srsqrt(var)
            nb = -mean * a
            av = jnp.full((NLANE,), a, dtype=jnp.float32)
            bv = jnp.full((NLANE,), nb, dtype=jnp.float32)
            for k in range(NVR):
                obuf[i, pl.ds(NLANE * k, NLANE)] = (e[k] * av + bv) * g[k] + bt[k]

    for d in gather(0, 0):
        d.start()

    def outer(gi, carry):
        for b in range(2):
            c = gi * 2 + b

            @pl.when(c <= NCHUNK - 2)
            def _():
                for d in gather(c + 1, 1 - b):
                    d.start()

            for d in gather(c, b):
                d.wait()

            @pl.when(c >= 2)
            def _():
                store(c - 2, b).wait()

            compute_chunk(pbufs[b], obufs[b])
            store(c, b).start()
        return carry

    lax.fori_loop(0, NCHUNK // 2, outer, 0)
    store(NCHUNK - 2, 0).wait()
    store(NCHUNK - 1, 1).wait()


@jax.jit
def _sc_embed(idx, table, pos_flat, gamma, beta):
    mesh = plsc.VectorSubcoreMesh(core_axis_name="c", subcore_axis_name="s")
    fn = pl.kernel(
        _body,
        out_type=jax.ShapeDtypeStruct((B, S, D), jnp.float32),
        mesh=mesh,
        compiler_params=pltpu.CompilerParams(
            needs_layout_passes=False, use_tc_tiling_on_sc=False
        ),
        scratch_types=[
            pltpu.VMEM((NCHUNK * CHUNK,), jnp.int32),
            pltpu.VMEM((S * D,), jnp.float32),
            [pltpu.VMEM((CHUNK, D), jnp.float32) for _ in range(2)],
            [pltpu.VMEM((CHUNK, D), jnp.float32) for _ in range(2)],
            pltpu.VMEM((D,), jnp.float32),
            pltpu.VMEM((D,), jnp.float32),
            [pltpu.SemaphoreType.DMA for _ in range(2)],
            [pltpu.SemaphoreType.DMA for _ in range(2)],
        ],
    )
    return fn(idx, table, pos_flat, gamma, beta)


def kernel(x, W_word, W_pos, gamma, beta):
    idx = x.reshape(N).astype(jnp.int32)
    pos_flat = W_pos.reshape(S * D)
    return _sc_embed(idx, W_word, pos_flat,
                     gamma.astype(jnp.float32), beta.astype(jnp.float32))
